# baseline (speedup 1.0000x reference)
import functools

import jax
import jax.numpy as jnp
from jax import lax
from jax.experimental import pallas as pl
from jax.experimental.pallas import tpu as pltpu
from jax.experimental.pallas import tpu_sc as plsc

N = 10000
E = 320000
D = 128
H1 = 32
H2 = 64
OUT = 64

NC = 2
NS = 16
NW = NC * NS
CHUNK = 128
NBUF = 4
NCHUNK = 80
EPAD = NW * NCHUNK * CHUNK
NPAD = 10240
RPT = NPAD // NS

ROWBLK = 1024
GRID = NPAD // ROWBLK


def _seg_mesh():
    return plsc.VectorSubcoreMesh(
        core_axis_name="c", subcore_axis_name="s", num_cores=NC, num_subcores=NS
    )


def _sc_segment_sum(table, sidx, didx, zeros, width):
    @functools.partial(
        pl.kernel,
        out_type=jax.ShapeDtypeStruct((NC, NPAD, width), jnp.float32),
        mesh=_seg_mesh(),
        compiler_params=pltpu.CompilerParams(use_tc_tiling_on_sc=False),
        scratch_types=[
            pltpu.VMEM((NCHUNK, CHUNK), jnp.int32),
            pltpu.VMEM((NCHUNK, CHUNK), jnp.int32),
            [pltpu.VMEM((CHUNK, width), jnp.float32) for _ in range(2)],
            pltpu.VMEM_SHARED((NPAD, width), jnp.float32),
            [pltpu.SemaphoreType.DMA for _ in range(2)],
            [pltpu.SemaphoreType.DMA for _ in range(2)],
        ],
    )
    def k(table_hbm, sidx_hbm, didx_hbm, zeros_hbm, out_hbm,
          sidx_v, didx_v, rows_v, acc, gsem, ssem):
        cid = lax.axis_index("c")
        sid = lax.axis_index("s")
        g = cid * NS + sid
        pltpu.sync_copy(zeros_hbm, acc.at[pl.ds(sid * RPT, RPT)])
        pltpu.sync_copy(sidx_hbm.at[g], sidx_v)
        pltpu.sync_copy(didx_hbm.at[g], didx_v)
        plsc.subcore_barrier()

        def gather(j, b):
            pltpu.async_copy(table_hbm.at[sidx_v.at[j]], rows_v[b], gsem[b])

        def gwait(j, b):
            pltpu.make_async_copy(table_hbm.at[sidx_v.at[j]],
                                  rows_v[b], gsem[b]).wait()

        def scat(j, b):
            pltpu.async_copy(rows_v[b], acc.at[didx_v.at[j]], ssem[b],
                             add=True)

        def swait(j, b):
            pltpu.make_async_copy(rows_v[b], acc.at[didx_v.at[j]],
                                  ssem[b]).wait()

        gather(0, 0)
        gwait(0, 0)
        gather(1, 1)
        scat(0, 0)

        def body(it, carry):
            j1 = 2 * it + 1
            j2 = j1 + 1
            gwait(j1, 1)
            swait(j1 - 1, 0)
            gather(j2, 0)
            scat(j1, 1)
            gwait(j2, 0)
            swait(j2 - 1, 1)
            gather(j2 + 1, 1)
            scat(j2, 0)
            return carry

        lax.fori_loop(0, (NCHUNK - 2) // 2, body, 0)
        gwait(NCHUNK - 1, 1)
        swait(NCHUNK - 2, 0)
        scat(NCHUNK - 1, 1)
        swait(NCHUNK - 1, 1)
        plsc.subcore_barrier()
        pltpu.sync_copy(acc.at[pl.ds(sid * RPT, RPT)],
                        out_hbm.at[cid, pl.ds(sid * RPT, RPT)])

    return k(table, sidx, didx, zeros)


def _sc_degree(didx, ones_rows, zeros):
    @functools.partial(
        pl.kernel,
        out_type=jax.ShapeDtypeStruct((NC, NPAD, 8), jnp.float32),
        mesh=_seg_mesh(),
        compiler_params=pltpu.CompilerParams(use_tc_tiling_on_sc=False),
        scratch_types=[
            pltpu.VMEM((NCHUNK, CHUNK), jnp.int32),
            pltpu.VMEM((CHUNK, 8), jnp.float32),
            pltpu.VMEM_SHARED((NPAD, 8), jnp.float32),
        ],
    )
    def k(didx_hbm, ones_hbm, zeros_hbm, out_hbm, didx_v, ones_v, acc):
        cid = lax.axis_index("c")
        sid = lax.axis_index("s")
        g = cid * NS + sid
        pltpu.sync_copy(zeros_hbm, acc.at[pl.ds(sid * RPT, RPT)])
        pltpu.sync_copy(didx_hbm.at[g], didx_v)
        pltpu.sync_copy(ones_hbm, ones_v)
        plsc.subcore_barrier()

        def body(j, carry):
            pltpu.sync_copy(ones_v, acc.at[didx_v.at[j]], add=True)
            return carry

        lax.fori_loop(0, NCHUNK, body, 0)
        plsc.subcore_barrier()
        pltpu.sync_copy(acc.at[pl.ds(sid * RPT, RPT)],
                        out_hbm.at[cid, pl.ds(sid * RPT, RPT)])

    return k(didx, ones_rows, zeros)


def _row_spec(w):
    return pl.BlockSpec((ROWBLK, w), lambda i: (i, 0))


def _part_spec(w):
    return pl.BlockSpec((NC, ROWBLK, w), lambda i: (0, i, 0))


def _full_spec(a):
    return pl.BlockSpec(a.shape, lambda i: tuple(0 for _ in a.shape))


def _tc1(degp, Wa0, ba0, Wb0, bb0):
    def body(degp_ref, wa, ba, wb, bb, x1a_ref, dinv_ref):
        d = degp_ref[0, :, 0:1] + degp_ref[1, :, 0:1]
        z = jnp.maximum((d + 1.0) * wa[...] + ba[...], 0.0)
        x1a = jnp.dot(z, wb[...], preferred_element_type=jnp.float32) + bb[...]
        x1a_ref[...] = jnp.maximum(x1a, 0.0)
        dinv_ref[...] = 1.0 / jnp.maximum(d, 1.0)

    return pl.pallas_call(
        body,
        grid=(GRID,),
        in_specs=[_part_spec(8), _full_spec(Wa0), _full_spec(ba0),
                  _full_spec(Wb0), _full_spec(bb0)],
        out_specs=[_row_spec(H1), _row_spec(1)],
        out_shape=[jax.ShapeDtypeStruct((NPAD, H1), jnp.float32),
                   jax.ShapeDtypeStruct((NPAD, 1), jnp.float32)],
    )(degp, Wa0, ba0, Wb0, bb0)


def _tc2(x1a, agg1p, xp, Wa1, ba1, Wb1, bb1, Wn0x, Wn0g, Ws0x, Ws0g):
    def body(x1a_ref, aggp_ref, x_ref, wa, ba, wb, bb, wnx, wng, wsx, wsg,
             y0_ref, s0_ref):
        agg1 = aggp_ref[0] + aggp_ref[1]
        z = x1a_ref[...] + agg1
        z = jnp.maximum(jnp.dot(z, wa[...], preferred_element_type=jnp.float32) + ba[...], 0.0)
        x1b = jnp.dot(z, wb[...], preferred_element_type=jnp.float32) + bb[...]
        xv = x_ref[...]
        y0_ref[...] = (jnp.dot(xv, wnx[...], preferred_element_type=jnp.float32)
                       + jnp.dot(x1b, wng[...], preferred_element_type=jnp.float32))
        s0_ref[...] = (jnp.dot(xv, wsx[...], preferred_element_type=jnp.float32)
                       + jnp.dot(x1b, wsg[...], preferred_element_type=jnp.float32))

    return pl.pallas_call(
        body,
        grid=(GRID,),
        in_specs=[_row_spec(H1), _part_spec(H1), _row_spec(D),
                  _full_spec(Wa1), _full_spec(ba1), _full_spec(Wb1), _full_spec(bb1),
                  _full_spec(Wn0x), _full_spec(Wn0g), _full_spec(Ws0x), _full_spec(Ws0g)],
        out_specs=[_row_spec(H2), _row_spec(H2)],
        out_shape=[jax.ShapeDtypeStruct((NPAD, H2), jnp.float32),
                   jax.ShapeDtypeStruct((NPAD, H2), jnp.float32)],
    )(x1a, agg1p, xp, Wa1, ba1, Wb1, bb1, Wn0x, Wn0g, Ws0x, Ws0g)


def _tc3(s0, aggy0p, dinv, Wn1, Ws1, b0, b1):
    def body(s0_ref, aggp_ref, dinv_ref, wn, ws, b0r, b1r, y1_ref, s1_ref):
        agg = aggp_ref[0] + aggp_ref[1]
        h = jnp.maximum(s0_ref[...] + agg * dinv_ref[...] + b0r[...], 0.0)
        y1_ref[...] = jnp.dot(h, wn[...], preferred_element_type=jnp.float32)
        s1_ref[...] = jnp.dot(h, ws[...], preferred_element_type=jnp.float32) + b1r[...]

    return pl.pallas_call(
        body,
        grid=(GRID,),
        in_specs=[_row_spec(H2), _part_spec(H2), _row_spec(1),
                  _full_spec(Wn1), _full_spec(Ws1), _full_spec(b0), _full_spec(b1)],
        out_specs=[_row_spec(OUT), _row_spec(OUT)],
        out_shape=[jax.ShapeDtypeStruct((NPAD, OUT), jnp.float32),
                   jax.ShapeDtypeStruct((NPAD, OUT), jnp.float32)],
    )(s0, aggy0p, dinv, Wn1, Ws1, b0, b1)


def _tc4(s1, aggy1p, dinv):
    def body(s1_ref, aggp_ref, dinv_ref, out_ref):
        agg = aggp_ref[0] + aggp_ref[1]
        out_ref[...] = s1_ref[...] + agg * dinv_ref[...]

    return pl.pallas_call(
        body,
        grid=(GRID,),
        in_specs=[_row_spec(OUT), _part_spec(OUT), _row_spec(1)],
        out_specs=_row_spec(OUT),
        out_shape=jax.ShapeDtypeStruct((NPAD, OUT), jnp.float32),
    )(s1, aggy1p, dinv)


def kernel(x, edge_index, gin_Wa0, gin_ba0, gin_Wb0, gin_bb0, gin_Wa1,
           gin_ba1, gin_Wb1, gin_bb1, sage_Ws0, sage_Wn0, sage_b0,
           sage_Ws1, sage_Wn1, sage_b1):
    src = edge_index[0]
    dst = edge_index[1]
    pad = EPAD - E
    srcp = jnp.concatenate([src, jnp.zeros((pad,), jnp.int32)])
    dstp = jnp.concatenate([dst, jnp.full((pad,), N, jnp.int32)])
    sidx = srcp.reshape(NW, NCHUNK, CHUNK)
    didx = dstp.reshape(NW, NCHUNK, CHUNK)

    zeros8 = jnp.zeros((RPT, 8), jnp.float32)
    zeros32 = jnp.zeros((RPT, H1), jnp.float32)
    zeros64 = jnp.zeros((RPT, H2), jnp.float32)
    ones_rows = jnp.ones((CHUNK, 8), jnp.float32)

    xp = jnp.concatenate([x, jnp.zeros((NPAD - N, D), jnp.float32)])
    Wa0 = gin_Wa0.reshape(1, H1)
    ba0 = gin_ba0.reshape(1, H1)
    bb0 = gin_bb0.reshape(1, H1)
    ba1 = gin_ba1.reshape(1, H1)
    bb1 = gin_bb1.reshape(1, H1)
    b0 = sage_b0.reshape(1, H2)
    b1 = sage_b1.reshape(1, OUT)
    Wn0x, Wn0g = sage_Wn0[:D], sage_Wn0[D:]
    Ws0x, Ws0g = sage_Ws0[:D], sage_Ws0[D:]

    degp = _sc_degree(didx, ones_rows, zeros8)
    x1a, dinv = _tc1(degp, Wa0, ba0, gin_Wb0, bb0)
    agg1p = _sc_segment_sum(x1a, sidx, didx, zeros32, H1)
    y0, s0 = _tc2(x1a, agg1p, xp, gin_Wa1, ba1, gin_Wb1, bb1,
                  Wn0x, Wn0g, Ws0x, Ws0g)
    aggy0p = _sc_segment_sum(y0, sidx, didx, zeros64, H2)
    y1, s1 = _tc3(s0, aggy0p, dinv, sage_Wn1, sage_Ws1, b0, b1)
    aggy1p = _sc_segment_sum(y1, sidx, didx, zeros64, H2)
    out = _tc4(s1, aggy1p, dinv)
    return out[:N]


# speedup vs baseline: 1.8573x; 1.8573x over previous
import functools

import jax
import jax.numpy as jnp
from jax import lax
from jax.experimental import pallas as pl
from jax.experimental.pallas import tpu as pltpu
from jax.experimental.pallas import tpu_sc as plsc

N = 10000
E = 320000
D = 128
H1 = 32
H2 = 64
OUT = 64

NC = 2
NS = 16
NW = NC * NS
CHUNK = 128
NBUF = 4
NCHUNK = 80
EPAD = NW * NCHUNK * CHUNK
NPAD = 10240
RPT = NPAD // NS

ROWBLK = 1024
GRID = NPAD // ROWBLK


def _seg_mesh():
    return plsc.VectorSubcoreMesh(
        core_axis_name="c", subcore_axis_name="s", num_cores=NC, num_subcores=NS
    )


def _sc_segment_sum(table, sidx, didx, zeros, width):
    @functools.partial(
        pl.kernel,
        out_type=jax.ShapeDtypeStruct((NC, NPAD, width), jnp.float32),
        mesh=_seg_mesh(),
        compiler_params=pltpu.CompilerParams(use_tc_tiling_on_sc=False),
        scratch_types=[
            pltpu.VMEM((NCHUNK, CHUNK), jnp.int32),
            pltpu.VMEM((NCHUNK, CHUNK), jnp.int32),
            pltpu.VMEM((CHUNK, width), jnp.float32),
            pltpu.VMEM_SHARED((NPAD, width), jnp.float32),
            pltpu.VMEM_SHARED((NPAD, width), jnp.float32),
            pltpu.SemaphoreType.DMA,
        ],
    )
    def k(table_hbm, sidx_hbm, didx_hbm, zeros_hbm, out_hbm,
          sidx_v, didx_v, rows_v, acc, table_s, sem):
        cid = lax.axis_index("c")
        sid = lax.axis_index("s")
        g = cid * NS + sid
        pltpu.sync_copy(zeros_hbm, acc.at[pl.ds(sid * RPT, RPT)])
        pltpu.sync_copy(table_hbm.at[pl.ds(sid * RPT, RPT)],
                        table_s.at[pl.ds(sid * RPT, RPT)])
        pltpu.sync_copy(sidx_hbm.at[g], sidx_v)
        pltpu.sync_copy(didx_hbm.at[g], didx_v)
        plsc.subcore_barrier()

        def body(j, carry):
            pltpu.async_copy(table_s.at[sidx_v.at[j]], rows_v, sem).wait()
            pltpu.sync_copy(rows_v, acc.at[didx_v.at[j]], add=True)
            return carry

        lax.fori_loop(0, NCHUNK, body, 0)
        plsc.subcore_barrier()
        pltpu.sync_copy(acc.at[pl.ds(sid * RPT, RPT)],
                        out_hbm.at[cid, pl.ds(sid * RPT, RPT)])

    return k(table, sidx, didx, zeros)


def _sc_degree(didx, ones_rows, zeros):
    @functools.partial(
        pl.kernel,
        out_type=jax.ShapeDtypeStruct((NC, NPAD, 8), jnp.float32),
        mesh=_seg_mesh(),
        compiler_params=pltpu.CompilerParams(use_tc_tiling_on_sc=False),
        scratch_types=[
            pltpu.VMEM((NCHUNK, CHUNK), jnp.int32),
            pltpu.VMEM((CHUNK, 8), jnp.float32),
            pltpu.VMEM_SHARED((NPAD, 8), jnp.float32),
        ],
    )
    def k(didx_hbm, ones_hbm, zeros_hbm, out_hbm, didx_v, ones_v, acc):
        cid = lax.axis_index("c")
        sid = lax.axis_index("s")
        g = cid * NS + sid
        pltpu.sync_copy(zeros_hbm, acc.at[pl.ds(sid * RPT, RPT)])
        pltpu.sync_copy(didx_hbm.at[g], didx_v)
        pltpu.sync_copy(ones_hbm, ones_v)
        plsc.subcore_barrier()

        def body(j, carry):
            pltpu.sync_copy(ones_v, acc.at[didx_v.at[j]], add=True)
            return carry

        lax.fori_loop(0, NCHUNK, body, 0)
        plsc.subcore_barrier()
        pltpu.sync_copy(acc.at[pl.ds(sid * RPT, RPT)],
                        out_hbm.at[cid, pl.ds(sid * RPT, RPT)])

    return k(didx, ones_rows, zeros)


def _row_spec(w):
    return pl.BlockSpec((ROWBLK, w), lambda i: (i, 0))


def _part_spec(w):
    return pl.BlockSpec((NC, ROWBLK, w), lambda i: (0, i, 0))


def _full_spec(a):
    return pl.BlockSpec(a.shape, lambda i: tuple(0 for _ in a.shape))


def _tc1(degp, Wa0, ba0, Wb0, bb0):
    def body(degp_ref, wa, ba, wb, bb, x1a_ref, dinv_ref):
        d = degp_ref[0, :, 0:1] + degp_ref[1, :, 0:1]
        z = jnp.maximum((d + 1.0) * wa[...] + ba[...], 0.0)
        x1a = jnp.dot(z, wb[...], preferred_element_type=jnp.float32) + bb[...]
        x1a_ref[...] = jnp.maximum(x1a, 0.0)
        dinv_ref[...] = 1.0 / jnp.maximum(d, 1.0)

    return pl.pallas_call(
        body,
        grid=(GRID,),
        in_specs=[_part_spec(8), _full_spec(Wa0), _full_spec(ba0),
                  _full_spec(Wb0), _full_spec(bb0)],
        out_specs=[_row_spec(H1), _row_spec(1)],
        out_shape=[jax.ShapeDtypeStruct((NPAD, H1), jnp.float32),
                   jax.ShapeDtypeStruct((NPAD, 1), jnp.float32)],
    )(degp, Wa0, ba0, Wb0, bb0)


def _tc2(x1a, agg1p, xp, Wa1, ba1, Wb1, bb1, Wn0x, Wn0g, Ws0x, Ws0g):
    def body(x1a_ref, aggp_ref, x_ref, wa, ba, wb, bb, wnx, wng, wsx, wsg,
             y0_ref, s0_ref):
        agg1 = aggp_ref[0] + aggp_ref[1]
        z = x1a_ref[...] + agg1
        z = jnp.maximum(jnp.dot(z, wa[...], preferred_element_type=jnp.float32) + ba[...], 0.0)
        x1b = jnp.dot(z, wb[...], preferred_element_type=jnp.float32) + bb[...]
        xv = x_ref[...]
        y0_ref[...] = (jnp.dot(xv, wnx[...], preferred_element_type=jnp.float32)
                       + jnp.dot(x1b, wng[...], preferred_element_type=jnp.float32))
        s0_ref[...] = (jnp.dot(xv, wsx[...], preferred_element_type=jnp.float32)
                       + jnp.dot(x1b, wsg[...], preferred_element_type=jnp.float32))

    return pl.pallas_call(
        body,
        grid=(GRID,),
        in_specs=[_row_spec(H1), _part_spec(H1), _row_spec(D),
                  _full_spec(Wa1), _full_spec(ba1), _full_spec(Wb1), _full_spec(bb1),
                  _full_spec(Wn0x), _full_spec(Wn0g), _full_spec(Ws0x), _full_spec(Ws0g)],
        out_specs=[_row_spec(H2), _row_spec(H2)],
        out_shape=[jax.ShapeDtypeStruct((NPAD, H2), jnp.float32),
                   jax.ShapeDtypeStruct((NPAD, H2), jnp.float32)],
    )(x1a, agg1p, xp, Wa1, ba1, Wb1, bb1, Wn0x, Wn0g, Ws0x, Ws0g)


def _tc3(s0, aggy0p, dinv, Wn1, Ws1, b0, b1):
    def body(s0_ref, aggp_ref, dinv_ref, wn, ws, b0r, b1r, y1_ref, s1_ref):
        agg = aggp_ref[0] + aggp_ref[1]
        h = jnp.maximum(s0_ref[...] + agg * dinv_ref[...] + b0r[...], 0.0)
        y1_ref[...] = jnp.dot(h, wn[...], preferred_element_type=jnp.float32)
        s1_ref[...] = jnp.dot(h, ws[...], preferred_element_type=jnp.float32) + b1r[...]

    return pl.pallas_call(
        body,
        grid=(GRID,),
        in_specs=[_row_spec(H2), _part_spec(H2), _row_spec(1),
                  _full_spec(Wn1), _full_spec(Ws1), _full_spec(b0), _full_spec(b1)],
        out_specs=[_row_spec(OUT), _row_spec(OUT)],
        out_shape=[jax.ShapeDtypeStruct((NPAD, OUT), jnp.float32),
                   jax.ShapeDtypeStruct((NPAD, OUT), jnp.float32)],
    )(s0, aggy0p, dinv, Wn1, Ws1, b0, b1)


def _tc4(s1, aggy1p, dinv):
    def body(s1_ref, aggp_ref, dinv_ref, out_ref):
        agg = aggp_ref[0] + aggp_ref[1]
        out_ref[...] = s1_ref[...] + agg * dinv_ref[...]

    return pl.pallas_call(
        body,
        grid=(GRID,),
        in_specs=[_row_spec(OUT), _part_spec(OUT), _row_spec(1)],
        out_specs=_row_spec(OUT),
        out_shape=jax.ShapeDtypeStruct((NPAD, OUT), jnp.float32),
    )(s1, aggy1p, dinv)


def kernel(x, edge_index, gin_Wa0, gin_ba0, gin_Wb0, gin_bb0, gin_Wa1,
           gin_ba1, gin_Wb1, gin_bb1, sage_Ws0, sage_Wn0, sage_b0,
           sage_Ws1, sage_Wn1, sage_b1):
    src = edge_index[0]
    dst = edge_index[1]
    pad = EPAD - E
    srcp = jnp.concatenate([src, jnp.zeros((pad,), jnp.int32)])
    dstp = jnp.concatenate([dst, jnp.full((pad,), N, jnp.int32)])
    sidx = srcp.reshape(NW, NCHUNK, CHUNK)
    didx = dstp.reshape(NW, NCHUNK, CHUNK)

    zeros8 = jnp.zeros((RPT, 8), jnp.float32)
    zeros32 = jnp.zeros((RPT, H1), jnp.float32)
    zeros64 = jnp.zeros((RPT, H2), jnp.float32)
    ones_rows = jnp.ones((CHUNK, 8), jnp.float32)

    xp = jnp.concatenate([x, jnp.zeros((NPAD - N, D), jnp.float32)])
    Wa0 = gin_Wa0.reshape(1, H1)
    ba0 = gin_ba0.reshape(1, H1)
    bb0 = gin_bb0.reshape(1, H1)
    ba1 = gin_ba1.reshape(1, H1)
    bb1 = gin_bb1.reshape(1, H1)
    b0 = sage_b0.reshape(1, H2)
    b1 = sage_b1.reshape(1, OUT)
    Wn0x, Wn0g = sage_Wn0[:D], sage_Wn0[D:]
    Ws0x, Ws0g = sage_Ws0[:D], sage_Ws0[D:]

    degp = _sc_degree(didx, ones_rows, zeros8)
    x1a, dinv = _tc1(degp, Wa0, ba0, gin_Wb0, bb0)
    agg1p = _sc_segment_sum(x1a, sidx, didx, zeros32, H1)
    y0, s0 = _tc2(x1a, agg1p, xp, gin_Wa1, ba1, gin_Wb1, bb1,
                  Wn0x, Wn0g, Ws0x, Ws0g)
    aggy0p = _sc_segment_sum(y0, sidx, didx, zeros64, H2)
    y1, s1 = _tc3(s0, aggy0p, dinv, sage_Wn1, sage_Ws1, b0, b1)
    aggy1p = _sc_segment_sum(y1, sidx, didx, zeros64, H2)
    out = _tc4(s1, aggy1p, dinv)
    return out[:N]


# speedup vs baseline: 2.2214x; 1.1960x over previous
import functools

import jax
import jax.numpy as jnp
from jax import lax
from jax.experimental import pallas as pl
from jax.experimental.pallas import tpu as pltpu
from jax.experimental.pallas import tpu_sc as plsc

N = 10000
E = 320000
D = 128
H1 = 32
H2 = 64
OUT = 64

NC = 2
NS = 16
NW = NC * NS
CHUNK = 128
NBUF = 4
NCHUNK = 80
EPAD = NW * NCHUNK * CHUNK
NPAD = 10240
RPT = NPAD // NS

ROWBLK = 1024
GRID = NPAD // ROWBLK


def _seg_mesh():
    return plsc.VectorSubcoreMesh(
        core_axis_name="c", subcore_axis_name="s", num_cores=NC, num_subcores=NS
    )


def _sc_segment_sum(table, sidx, didx, zeros, width):
    @functools.partial(
        pl.kernel,
        out_type=jax.ShapeDtypeStruct((NC, NPAD, width), jnp.float32),
        mesh=_seg_mesh(),
        compiler_params=pltpu.CompilerParams(use_tc_tiling_on_sc=False),
        scratch_types=[
            pltpu.VMEM((NCHUNK, CHUNK), jnp.int32),
            pltpu.VMEM((NCHUNK, CHUNK), jnp.int32),
            [pltpu.VMEM((CHUNK, width), jnp.float32) for _ in range(2)],
            pltpu.VMEM_SHARED((NPAD, width), jnp.float32),
            pltpu.VMEM_SHARED((NPAD, width), jnp.float32),
            [pltpu.SemaphoreType.DMA for _ in range(2)],
            [pltpu.SemaphoreType.DMA for _ in range(2)],
        ],
    )
    def k(table_hbm, sidx_hbm, didx_hbm, zeros_hbm, out_hbm,
          sidx_v, didx_v, rows_v, acc, table_s, gsem, ssem):
        cid = lax.axis_index("c")
        sid = lax.axis_index("s")
        g = cid * NS + sid
        pltpu.sync_copy(zeros_hbm, acc.at[pl.ds(sid * RPT, RPT)])
        pltpu.sync_copy(table_hbm.at[pl.ds(sid * RPT, RPT)],
                        table_s.at[pl.ds(sid * RPT, RPT)])
        pltpu.sync_copy(sidx_hbm.at[g], sidx_v)
        pltpu.sync_copy(didx_hbm.at[g], didx_v)
        plsc.subcore_barrier()

        def gather(j, b):
            pltpu.async_copy(table_s.at[sidx_v.at[j]], rows_v[b], gsem[b])

        def gwait(j, b):
            pltpu.make_async_copy(table_s.at[sidx_v.at[j]],
                                  rows_v[b], gsem[b]).wait()

        def scat(j, b):
            pltpu.async_copy(rows_v[b], acc.at[didx_v.at[j]], ssem[b],
                             add=True)

        def swait(j, b):
            pltpu.make_async_copy(rows_v[b], acc.at[didx_v.at[j]],
                                  ssem[b]).wait()

        gather(0, 0)
        gwait(0, 0)
        gather(1, 1)
        scat(0, 0)

        def body(it, carry):
            j1 = 2 * it + 1
            j2 = j1 + 1
            gwait(j1, 1)
            swait(j1 - 1, 0)
            gather(j2, 0)
            scat(j1, 1)
            gwait(j2, 0)
            swait(j2 - 1, 1)
            gather(j2 + 1, 1)
            scat(j2, 0)
            return carry

        lax.fori_loop(0, (NCHUNK - 2) // 2, body, 0)
        gwait(NCHUNK - 1, 1)
        swait(NCHUNK - 2, 0)
        scat(NCHUNK - 1, 1)
        swait(NCHUNK - 1, 1)
        plsc.subcore_barrier()
        pltpu.sync_copy(acc.at[pl.ds(sid * RPT, RPT)],
                        out_hbm.at[cid, pl.ds(sid * RPT, RPT)])

    return k(table, sidx, didx, zeros)


def _sc_degree(didx, ones_rows, zeros):
    @functools.partial(
        pl.kernel,
        out_type=jax.ShapeDtypeStruct((NC, NPAD, 8), jnp.float32),
        mesh=_seg_mesh(),
        compiler_params=pltpu.CompilerParams(use_tc_tiling_on_sc=False),
        scratch_types=[
            pltpu.VMEM((NCHUNK, CHUNK), jnp.int32),
            pltpu.VMEM((CHUNK, 8), jnp.float32),
            pltpu.VMEM_SHARED((NPAD, 8), jnp.float32),
        ],
    )
    def k(didx_hbm, ones_hbm, zeros_hbm, out_hbm, didx_v, ones_v, acc):
        cid = lax.axis_index("c")
        sid = lax.axis_index("s")
        g = cid * NS + sid
        pltpu.sync_copy(zeros_hbm, acc.at[pl.ds(sid * RPT, RPT)])
        pltpu.sync_copy(didx_hbm.at[g], didx_v)
        pltpu.sync_copy(ones_hbm, ones_v)
        plsc.subcore_barrier()

        def body(j, carry):
            pltpu.sync_copy(ones_v, acc.at[didx_v.at[j]], add=True)
            return carry

        lax.fori_loop(0, NCHUNK, body, 0)
        plsc.subcore_barrier()
        pltpu.sync_copy(acc.at[pl.ds(sid * RPT, RPT)],
                        out_hbm.at[cid, pl.ds(sid * RPT, RPT)])

    return k(didx, ones_rows, zeros)


def _row_spec(w):
    return pl.BlockSpec((ROWBLK, w), lambda i: (i, 0))


def _part_spec(w):
    return pl.BlockSpec((NC, ROWBLK, w), lambda i: (0, i, 0))


def _full_spec(a):
    return pl.BlockSpec(a.shape, lambda i: tuple(0 for _ in a.shape))


def _tc1(degp, Wa0, ba0, Wb0, bb0):
    def body(degp_ref, wa, ba, wb, bb, x1a_ref, dinv_ref):
        d = degp_ref[0, :, 0:1] + degp_ref[1, :, 0:1]
        z = jnp.maximum((d + 1.0) * wa[...] + ba[...], 0.0)
        x1a = jnp.dot(z, wb[...], preferred_element_type=jnp.float32) + bb[...]
        x1a_ref[...] = jnp.maximum(x1a, 0.0)
        dinv_ref[...] = 1.0 / jnp.maximum(d, 1.0)

    return pl.pallas_call(
        body,
        grid=(GRID,),
        in_specs=[_part_spec(8), _full_spec(Wa0), _full_spec(ba0),
                  _full_spec(Wb0), _full_spec(bb0)],
        out_specs=[_row_spec(H1), _row_spec(1)],
        out_shape=[jax.ShapeDtypeStruct((NPAD, H1), jnp.float32),
                   jax.ShapeDtypeStruct((NPAD, 1), jnp.float32)],
    )(degp, Wa0, ba0, Wb0, bb0)


def _tc2(x1a, agg1p, xp, Wa1, ba1, Wb1, bb1, Wn0x, Wn0g, Ws0x, Ws0g):
    def body(x1a_ref, aggp_ref, x_ref, wa, ba, wb, bb, wnx, wng, wsx, wsg,
             y0_ref, s0_ref):
        agg1 = aggp_ref[0] + aggp_ref[1]
        z = x1a_ref[...] + agg1
        z = jnp.maximum(jnp.dot(z, wa[...], preferred_element_type=jnp.float32) + ba[...], 0.0)
        x1b = jnp.dot(z, wb[...], preferred_element_type=jnp.float32) + bb[...]
        xv = x_ref[...]
        y0_ref[...] = (jnp.dot(xv, wnx[...], preferred_element_type=jnp.float32)
                       + jnp.dot(x1b, wng[...], preferred_element_type=jnp.float32))
        s0_ref[...] = (jnp.dot(xv, wsx[...], preferred_element_type=jnp.float32)
                       + jnp.dot(x1b, wsg[...], preferred_element_type=jnp.float32))

    return pl.pallas_call(
        body,
        grid=(GRID,),
        in_specs=[_row_spec(H1), _part_spec(H1), _row_spec(D),
                  _full_spec(Wa1), _full_spec(ba1), _full_spec(Wb1), _full_spec(bb1),
                  _full_spec(Wn0x), _full_spec(Wn0g), _full_spec(Ws0x), _full_spec(Ws0g)],
        out_specs=[_row_spec(H2), _row_spec(H2)],
        out_shape=[jax.ShapeDtypeStruct((NPAD, H2), jnp.float32),
                   jax.ShapeDtypeStruct((NPAD, H2), jnp.float32)],
    )(x1a, agg1p, xp, Wa1, ba1, Wb1, bb1, Wn0x, Wn0g, Ws0x, Ws0g)


def _tc3(s0, aggy0p, dinv, Wn1, Ws1, b0, b1):
    def body(s0_ref, aggp_ref, dinv_ref, wn, ws, b0r, b1r, y1_ref, s1_ref):
        agg = aggp_ref[0] + aggp_ref[1]
        h = jnp.maximum(s0_ref[...] + agg * dinv_ref[...] + b0r[...], 0.0)
        y1_ref[...] = jnp.dot(h, wn[...], preferred_element_type=jnp.float32)
        s1_ref[...] = jnp.dot(h, ws[...], preferred_element_type=jnp.float32) + b1r[...]

    return pl.pallas_call(
        body,
        grid=(GRID,),
        in_specs=[_row_spec(H2), _part_spec(H2), _row_spec(1),
                  _full_spec(Wn1), _full_spec(Ws1), _full_spec(b0), _full_spec(b1)],
        out_specs=[_row_spec(OUT), _row_spec(OUT)],
        out_shape=[jax.ShapeDtypeStruct((NPAD, OUT), jnp.float32),
                   jax.ShapeDtypeStruct((NPAD, OUT), jnp.float32)],
    )(s0, aggy0p, dinv, Wn1, Ws1, b0, b1)


def _tc4(s1, aggy1p, dinv):
    def body(s1_ref, aggp_ref, dinv_ref, out_ref):
        agg = aggp_ref[0] + aggp_ref[1]
        out_ref[...] = s1_ref[...] + agg * dinv_ref[...]

    return pl.pallas_call(
        body,
        grid=(GRID,),
        in_specs=[_row_spec(OUT), _part_spec(OUT), _row_spec(1)],
        out_specs=_row_spec(OUT),
        out_shape=jax.ShapeDtypeStruct((NPAD, OUT), jnp.float32),
    )(s1, aggy1p, dinv)


def kernel(x, edge_index, gin_Wa0, gin_ba0, gin_Wb0, gin_bb0, gin_Wa1,
           gin_ba1, gin_Wb1, gin_bb1, sage_Ws0, sage_Wn0, sage_b0,
           sage_Ws1, sage_Wn1, sage_b1):
    src = edge_index[0]
    dst = edge_index[1]
    pad = EPAD - E
    srcp = jnp.concatenate([src, jnp.zeros((pad,), jnp.int32)])
    dstp = jnp.concatenate([dst, jnp.full((pad,), N, jnp.int32)])
    sidx = srcp.reshape(NW, NCHUNK, CHUNK)
    didx = dstp.reshape(NW, NCHUNK, CHUNK)

    zeros8 = jnp.zeros((RPT, 8), jnp.float32)
    zeros32 = jnp.zeros((RPT, H1), jnp.float32)
    zeros64 = jnp.zeros((RPT, H2), jnp.float32)
    ones_rows = jnp.ones((CHUNK, 8), jnp.float32)

    xp = jnp.concatenate([x, jnp.zeros((NPAD - N, D), jnp.float32)])
    Wa0 = gin_Wa0.reshape(1, H1)
    ba0 = gin_ba0.reshape(1, H1)
    bb0 = gin_bb0.reshape(1, H1)
    ba1 = gin_ba1.reshape(1, H1)
    bb1 = gin_bb1.reshape(1, H1)
    b0 = sage_b0.reshape(1, H2)
    b1 = sage_b1.reshape(1, OUT)
    Wn0x, Wn0g = sage_Wn0[:D], sage_Wn0[D:]
    Ws0x, Ws0g = sage_Ws0[:D], sage_Ws0[D:]

    degp = _sc_degree(didx, ones_rows, zeros8)
    x1a, dinv = _tc1(degp, Wa0, ba0, gin_Wb0, bb0)
    agg1p = _sc_segment_sum(x1a, sidx, didx, zeros32, H1)
    y0, s0 = _tc2(x1a, agg1p, xp, gin_Wa1, ba1, gin_Wb1, bb1,
                  Wn0x, Wn0g, Ws0x, Ws0g)
    aggy0p = _sc_segment_sum(y0, sidx, didx, zeros64, H2)
    y1, s1 = _tc3(s0, aggy0p, dinv, sage_Wn1, sage_Ws1, b0, b1)
    aggy1p = _sc_segment_sum(y1, sidx, didx, zeros64, H2)
    out = _tc4(s1, aggy1p, dinv)
    return out[:N]


# speedup vs baseline: 2.2749x; 1.0241x over previous
import functools

import jax
import jax.numpy as jnp
from jax import lax
from jax.experimental import pallas as pl
from jax.experimental.pallas import tpu as pltpu
from jax.experimental.pallas import tpu_sc as plsc

N = 10000
E = 320000
D = 128
H1 = 32
H2 = 64
OUT = 64

NC = 2
NS = 16
NW = NC * NS
CHUNK = 128
NBUF = 4
NCHUNK = 80
EPAD = NW * NCHUNK * CHUNK
NPAD = 10240
RPT = NPAD // NS

ROWBLK = 1024
GRID = NPAD // ROWBLK


def _seg_mesh():
    return plsc.VectorSubcoreMesh(
        core_axis_name="c", subcore_axis_name="s", num_cores=NC, num_subcores=NS
    )


def _sc_segment_sum(table, sidx, didx, zeros, width, chunk, nchunk):
    @functools.partial(
        pl.kernel,
        out_type=jax.ShapeDtypeStruct((NC, NPAD, width), jnp.float32),
        mesh=_seg_mesh(),
        compiler_params=pltpu.CompilerParams(use_tc_tiling_on_sc=False),
        scratch_types=[
            pltpu.VMEM((nchunk, chunk), jnp.int32),
            pltpu.VMEM((nchunk, chunk), jnp.int32),
            [pltpu.VMEM((chunk, width), jnp.float32) for _ in range(NBUF)],
            pltpu.VMEM_SHARED((NPAD, width), jnp.float32),
            pltpu.VMEM_SHARED((NPAD, width), jnp.float32),
            [pltpu.SemaphoreType.DMA for _ in range(NBUF)],
            [pltpu.SemaphoreType.DMA for _ in range(NBUF)],
        ],
    )
    def k(table_hbm, sidx_hbm, didx_hbm, zeros_hbm, out_hbm,
          sidx_v, didx_v, rows_v, acc, table_s, gsem, ssem):
        cid = lax.axis_index("c")
        sid = lax.axis_index("s")
        g = cid * NS + sid
        pltpu.sync_copy(zeros_hbm, acc.at[pl.ds(sid * RPT, RPT)])
        pltpu.sync_copy(table_hbm.at[pl.ds(sid * RPT, RPT)],
                        table_s.at[pl.ds(sid * RPT, RPT)])
        pltpu.sync_copy(sidx_hbm.at[g], sidx_v)
        pltpu.sync_copy(didx_hbm.at[g], didx_v)
        plsc.subcore_barrier()

        def gather(j, b):
            pltpu.async_copy(table_s.at[sidx_v.at[j]], rows_v[b], gsem[b])

        def gwait(j, b):
            pltpu.make_async_copy(table_s.at[sidx_v.at[j]],
                                  rows_v[b], gsem[b]).wait()

        def scat(j, b):
            pltpu.async_copy(rows_v[b], acc.at[didx_v.at[j]], ssem[b],
                             add=True)

        def swait(j, b):
            pltpu.make_async_copy(rows_v[b], acc.at[didx_v.at[j]],
                                  ssem[b]).wait()

        gather(0, 0)
        for j in range(NBUF - 1):
            gwait(j, j % NBUF)
            gather(j + 1, (j + 1) % NBUF)
            scat(j, j % NBUF)

        def body(it, carry):
            for s in range(NBUF):
                b = (NBUF - 1 + s) % NBUF
                bn = (b + 1) % NBUF
                j = (NBUF - 1) + it * NBUF + s
                gwait(j, b)
                swait(j - (NBUF - 1), bn)
                gather(j + 1, bn)
                scat(j, b)
            return carry

        lax.fori_loop(0, (nchunk - NBUF) // NBUF, body, 0)
        jl = nchunk - 1
        bl = jl % NBUF
        gwait(jl, bl)
        swait(jl - (NBUF - 1), (bl + 1) % NBUF)
        scat(jl, bl)
        for t in range(NBUF - 1):
            jj = jl - (NBUF - 2) + t
            swait(jj, jj % NBUF)
        plsc.subcore_barrier()
        pltpu.sync_copy(acc.at[pl.ds(sid * RPT, RPT)],
                        out_hbm.at[cid, pl.ds(sid * RPT, RPT)])

    return k(table, sidx, didx, zeros)


def _sc_degree(didx, ones_rows, zeros):
    @functools.partial(
        pl.kernel,
        out_type=jax.ShapeDtypeStruct((NC, NPAD, 8), jnp.float32),
        mesh=_seg_mesh(),
        compiler_params=pltpu.CompilerParams(use_tc_tiling_on_sc=False),
        scratch_types=[
            pltpu.VMEM((NCHUNK, CHUNK), jnp.int32),
            pltpu.VMEM((CHUNK, 8), jnp.float32),
            pltpu.VMEM_SHARED((NPAD, 8), jnp.float32),
        ],
    )
    def k(didx_hbm, ones_hbm, zeros_hbm, out_hbm, didx_v, ones_v, acc):
        cid = lax.axis_index("c")
        sid = lax.axis_index("s")
        g = cid * NS + sid
        pltpu.sync_copy(zeros_hbm, acc.at[pl.ds(sid * RPT, RPT)])
        pltpu.sync_copy(didx_hbm.at[g], didx_v)
        pltpu.sync_copy(ones_hbm, ones_v)
        plsc.subcore_barrier()

        def body(j, carry):
            pltpu.sync_copy(ones_v, acc.at[didx_v.at[j]], add=True)
            return carry

        lax.fori_loop(0, NCHUNK, body, 0)
        plsc.subcore_barrier()
        pltpu.sync_copy(acc.at[pl.ds(sid * RPT, RPT)],
                        out_hbm.at[cid, pl.ds(sid * RPT, RPT)])

    return k(didx, ones_rows, zeros)


def _row_spec(w):
    return pl.BlockSpec((ROWBLK, w), lambda i: (i, 0))


def _part_spec(w):
    return pl.BlockSpec((NC, ROWBLK, w), lambda i: (0, i, 0))


def _full_spec(a):
    return pl.BlockSpec(a.shape, lambda i: tuple(0 for _ in a.shape))


def _tc1(degp, Wa0, ba0, Wb0, bb0):
    def body(degp_ref, wa, ba, wb, bb, x1a_ref, dinv_ref):
        d = degp_ref[0, :, 0:1] + degp_ref[1, :, 0:1]
        z = jnp.maximum((d + 1.0) * wa[...] + ba[...], 0.0)
        x1a = jnp.dot(z, wb[...], preferred_element_type=jnp.float32) + bb[...]
        x1a_ref[...] = jnp.maximum(x1a, 0.0)
        dinv_ref[...] = 1.0 / jnp.maximum(d, 1.0)

    return pl.pallas_call(
        body,
        grid=(GRID,),
        in_specs=[_part_spec(8), _full_spec(Wa0), _full_spec(ba0),
                  _full_spec(Wb0), _full_spec(bb0)],
        out_specs=[_row_spec(H1), _row_spec(1)],
        out_shape=[jax.ShapeDtypeStruct((NPAD, H1), jnp.float32),
                   jax.ShapeDtypeStruct((NPAD, 1), jnp.float32)],
    )(degp, Wa0, ba0, Wb0, bb0)


def _tc2(x1a, agg1p, xp, Wa1, ba1, Wb1, bb1, Wn0x, Wn0g, Ws0x, Ws0g):
    def body(x1a_ref, aggp_ref, x_ref, wa, ba, wb, bb, wnx, wng, wsx, wsg,
             y0_ref, s0_ref):
        agg1 = aggp_ref[0] + aggp_ref[1]
        z = x1a_ref[...] + agg1
        z = jnp.maximum(jnp.dot(z, wa[...], preferred_element_type=jnp.float32) + ba[...], 0.0)
        x1b = jnp.dot(z, wb[...], preferred_element_type=jnp.float32) + bb[...]
        xv = x_ref[...]
        y0_ref[...] = (jnp.dot(xv, wnx[...], preferred_element_type=jnp.float32)
                       + jnp.dot(x1b, wng[...], preferred_element_type=jnp.float32))
        s0_ref[...] = (jnp.dot(xv, wsx[...], preferred_element_type=jnp.float32)
                       + jnp.dot(x1b, wsg[...], preferred_element_type=jnp.float32))

    return pl.pallas_call(
        body,
        grid=(GRID,),
        in_specs=[_row_spec(H1), _part_spec(H1), _row_spec(D),
                  _full_spec(Wa1), _full_spec(ba1), _full_spec(Wb1), _full_spec(bb1),
                  _full_spec(Wn0x), _full_spec(Wn0g), _full_spec(Ws0x), _full_spec(Ws0g)],
        out_specs=[_row_spec(H2), _row_spec(H2)],
        out_shape=[jax.ShapeDtypeStruct((NPAD, H2), jnp.float32),
                   jax.ShapeDtypeStruct((NPAD, H2), jnp.float32)],
    )(x1a, agg1p, xp, Wa1, ba1, Wb1, bb1, Wn0x, Wn0g, Ws0x, Ws0g)


def _tc3(s0, aggy0p, dinv, Wn1, Ws1, b0, b1):
    def body(s0_ref, aggp_ref, dinv_ref, wn, ws, b0r, b1r, y1_ref, s1_ref):
        agg = aggp_ref[0] + aggp_ref[1]
        h = jnp.maximum(s0_ref[...] + agg * dinv_ref[...] + b0r[...], 0.0)
        y1_ref[...] = jnp.dot(h, wn[...], preferred_element_type=jnp.float32)
        s1_ref[...] = jnp.dot(h, ws[...], preferred_element_type=jnp.float32) + b1r[...]

    return pl.pallas_call(
        body,
        grid=(GRID,),
        in_specs=[_row_spec(H2), _part_spec(H2), _row_spec(1),
                  _full_spec(Wn1), _full_spec(Ws1), _full_spec(b0), _full_spec(b1)],
        out_specs=[_row_spec(OUT), _row_spec(OUT)],
        out_shape=[jax.ShapeDtypeStruct((NPAD, OUT), jnp.float32),
                   jax.ShapeDtypeStruct((NPAD, OUT), jnp.float32)],
    )(s0, aggy0p, dinv, Wn1, Ws1, b0, b1)


def _tc4(s1, aggy1p, dinv):
    def body(s1_ref, aggp_ref, dinv_ref, out_ref):
        agg = aggp_ref[0] + aggp_ref[1]
        out_ref[...] = s1_ref[...] + agg * dinv_ref[...]

    return pl.pallas_call(
        body,
        grid=(GRID,),
        in_specs=[_row_spec(OUT), _part_spec(OUT), _row_spec(1)],
        out_specs=_row_spec(OUT),
        out_shape=jax.ShapeDtypeStruct((NPAD, OUT), jnp.float32),
    )(s1, aggy1p, dinv)


def kernel(x, edge_index, gin_Wa0, gin_ba0, gin_Wb0, gin_bb0, gin_Wa1,
           gin_ba1, gin_Wb1, gin_bb1, sage_Ws0, sage_Wn0, sage_b0,
           sage_Ws1, sage_Wn1, sage_b1):
    src = edge_index[0]
    dst = edge_index[1]
    pad = EPAD - E
    srcp = jnp.concatenate([src, jnp.zeros((pad,), jnp.int32)])
    dstp = jnp.concatenate([dst, jnp.full((pad,), N, jnp.int32)])
    sidx = srcp.reshape(NW, NCHUNK, CHUNK)
    didx = dstp.reshape(NW, NCHUNK, CHUNK)
    sidx64 = srcp.reshape(NW, NCHUNK * 2, CHUNK // 2)
    didx64 = dstp.reshape(NW, NCHUNK * 2, CHUNK // 2)

    zeros8 = jnp.zeros((RPT, 8), jnp.float32)
    zeros32 = jnp.zeros((RPT, H1), jnp.float32)
    zeros64 = jnp.zeros((RPT, H2), jnp.float32)
    ones_rows = jnp.ones((CHUNK, 8), jnp.float32)

    xp = jnp.concatenate([x, jnp.zeros((NPAD - N, D), jnp.float32)])
    Wa0 = gin_Wa0.reshape(1, H1)
    ba0 = gin_ba0.reshape(1, H1)
    bb0 = gin_bb0.reshape(1, H1)
    ba1 = gin_ba1.reshape(1, H1)
    bb1 = gin_bb1.reshape(1, H1)
    b0 = sage_b0.reshape(1, H2)
    b1 = sage_b1.reshape(1, OUT)
    Wn0x, Wn0g = sage_Wn0[:D], sage_Wn0[D:]
    Ws0x, Ws0g = sage_Ws0[:D], sage_Ws0[D:]

    degp = _sc_degree(didx, ones_rows, zeros8)
    x1a, dinv = _tc1(degp, Wa0, ba0, gin_Wb0, bb0)
    agg1p = _sc_segment_sum(x1a, sidx, didx, zeros32, H1, CHUNK, NCHUNK)
    y0, s0 = _tc2(x1a, agg1p, xp, gin_Wa1, ba1, gin_Wb1, bb1,
                  Wn0x, Wn0g, Ws0x, Ws0g)
    aggy0p = _sc_segment_sum(y0, sidx64, didx64, zeros64, H2,
                             CHUNK // 2, NCHUNK * 2)
    y1, s1 = _tc3(s0, aggy0p, dinv, sage_Wn1, sage_Ws1, b0, b1)
    aggy1p = _sc_segment_sum(y1, sidx64, didx64, zeros64, H2,
                             CHUNK // 2, NCHUNK * 2)
    out = _tc4(s1, aggy1p, dinv)
    return out[:N]
